# baseline (speedup 1.0000x reference)
import functools

import jax
import jax.numpy as jnp
from jax.experimental import pallas as pl

E = 8
K = 2
D = 768
H = 256
C_EXP = 100
C_TOT = 800
N = 2048

_EPS = 2.220446049250313e-16


def _moe_kernel(x_ref, wg_ref, w1_ref, b1_ref, w2_ref, b2_ref, wm_ref, out_ref):
    xt = x_ref[:]
    t = xt.shape[0]
    logits = jnp.dot(xt, wg_ref[:], preferred_element_type=jnp.float32)

    eidx = jax.lax.broadcasted_iota(jnp.int32, (t, E), 1)
    m1 = jnp.max(logits, axis=1, keepdims=True)
    a1 = jnp.argmax(logits, axis=1)[:, None]
    oh1 = (eidx == a1)
    masked = jnp.where(oh1, -jnp.inf, logits)
    m2 = jnp.max(masked, axis=1, keepdims=True)
    a2 = jnp.argmax(masked, axis=1)[:, None]
    oh2 = (eidx == a2)

    e2 = jnp.exp(m2 - m1)
    denom = 1.0 + e2
    g1 = 1.0 / denom
    g2 = e2 / denom
    gates = jnp.where(oh1, g1, 0.0) + jnp.where(oh2, g2, 0.0)

    acc = jnp.zeros((t, C_TOT), dtype=jnp.float32)
    for e in range(E):
        h = jnp.dot(xt, w1_ref[e], preferred_element_type=jnp.float32)
        h = jnp.maximum(h + b1_ref[e][None, :], 0.0)
        o = jnp.dot(h, w2_ref[e], preferred_element_type=jnp.float32)
        o = o + b2_ref[e][None, :]
        m = jnp.dot(o, wm_ref[e], preferred_element_type=jnp.float32)
        acc = acc + gates[:, e][:, None] * m
    acc = jnp.where(acc == 0.0, jnp.float32(_EPS), acc)
    out_ref[:] = acc


@functools.partial(jax.jit, static_argnames=("interpret",))
def _moe(x, w_gate, W1, b1, W2, b2, Wm, interpret=False):
    T = 256
    grid = (N // T,)
    full = lambda *s: pl.BlockSpec(s, lambda i: (0,) * len(s))
    return pl.pallas_call(
        _moe_kernel,
        grid=grid,
        in_specs=[
            pl.BlockSpec((T, D), lambda i: (i, 0)),
            full(D, E),
            full(E, D, H),
            full(E, H),
            full(E, H, C_EXP),
            full(E, C_EXP),
            full(E, C_EXP, C_TOT),
        ],
        out_specs=pl.BlockSpec((T, C_TOT), lambda i: (i, 0)),
        out_shape=jax.ShapeDtypeStruct((N, C_TOT), jnp.float32),
        interpret=interpret,
    )(x, w_gate, W1, b1, W2, b2, Wm)


def kernel(x, labels, w_gate, W1, b1, W2, b2, Wm):
    return _moe(x, w_gate, W1, b1, W2, b2, Wm)
